# baseline (speedup 1.0000x reference)
import functools

import jax
import jax.numpy as jnp
from jax import lax
from jax.experimental import pallas as pl
from jax.experimental.pallas import tpu as pltpu
from jax.experimental.pallas import tpu_sc as plsc

VOCAB = 100000
DIM = 128
BATCH = 4096
HIST = 50
N = BATCH * HIST
NC, NS = 2, 16
NW = NC * NS
COLS = 128
CH = 64
NCH = HIST * 2
NBUF = 10


@functools.partial(
    pl.kernel,
    mesh=plsc.VectorSubcoreMesh(core_axis_name="c", subcore_axis_name="s"),
    out_type=jax.ShapeDtypeStruct((N, DIM), jnp.float32),
    scratch_types=[
        pltpu.VMEM((HIST, COLS), jnp.int32),
    ]
    + [pltpu.VMEM((CH, DIM), jnp.float32) for _ in range(NBUF)]
    + [pltpu.SemaphoreType.DMA for _ in range(2 * NBUF)],
)
def _emb_gather(xt_hbm, table_hbm, out_hbm, idx_v, *bufs_and_sems):
    rows = bufs_and_sems[:NBUF]
    gsem = bufs_and_sems[NBUF:2 * NBUF]
    wsem = bufs_and_sems[2 * NBUF:]
    wid = lax.axis_index("s") * NC + lax.axis_index("c")
    col = wid * COLS
    pltpu.sync_copy(xt_hbm.at[:, pl.ds(col, COLS)], idx_v)

    def fire_gather(h, b):
        half = b % 2
        pltpu.async_copy(
            table_hbm.at[idx_v.at[h, pl.ds(half * CH, CH)]],
            rows[b], gsem[b])

    def fire_write(h, b):
        half = b % 2
        pltpu.async_copy(
            rows[b], out_hbm.at[pl.ds(h * BATCH + col + half * CH, CH)],
            wsem[b])

    for b in range(NBUF):
        fire_gather(b // 2, b)

    def outer(t, carry):
        for b in range(NBUF):
            h = t * (NBUF // 2) + b // 2
            pltpu.make_async_copy(
                table_hbm.at[pl.ds(0, CH)], rows[b], gsem[b]).wait()
            fire_write(h, b)
            pltpu.make_async_copy(
                rows[b], out_hbm.at[pl.ds(0, CH)], wsem[b]).wait()
            fire_gather(h + NBUF // 2, b)
        return carry

    lax.fori_loop(0, NCH // NBUF - 1, outer, 0)

    for b in range(NBUF):
        h = (NCH - NBUF + b) // 2
        pltpu.make_async_copy(
            table_hbm.at[pl.ds(0, CH)], rows[b], gsem[b]).wait()
        half = b % 2
        pltpu.sync_copy(
            rows[b], out_hbm.at[pl.ds(h * BATCH + col + half * CH, CH)])


def kernel(x, table):
    xt = x.T.astype(jnp.int32)
    out = _emb_gather(xt, table)
    return out.reshape(HIST, BATCH, DIM).transpose(1, 0, 2)


# speedup vs baseline: 1.0030x; 1.0030x over previous
import functools

import jax
import jax.numpy as jnp
from jax import lax
from jax.experimental import pallas as pl
from jax.experimental.pallas import tpu as pltpu
from jax.experimental.pallas import tpu_sc as plsc

VOCAB = 100000
DIM = 128
BATCH = 4096
HIST = 50
N = BATCH * HIST
NC, NS = 2, 16
NW = NC * NS
CH = 128
NCH = HIST
NBUF = 5


@functools.partial(
    pl.kernel,
    mesh=plsc.VectorSubcoreMesh(core_axis_name="c", subcore_axis_name="s"),
    out_type=jax.ShapeDtypeStruct((N, DIM), jnp.float32),
    scratch_types=[
        pltpu.VMEM((NCH, CH), jnp.int32),
    ]
    + [pltpu.VMEM((CH, DIM), jnp.float32) for _ in range(NBUF)]
    + [pltpu.SemaphoreType.DMA for _ in range(2 * NBUF)],
)
def _emb_gather(xt_hbm, table_hbm, out_hbm, idx_v, *bufs_and_sems):
    rows = bufs_and_sems[:NBUF]
    gsem = bufs_and_sems[NBUF:2 * NBUF]
    wsem = bufs_and_sems[2 * NBUF:]
    wid = lax.axis_index("s") * NC + lax.axis_index("c")
    col = wid * CH
    pltpu.sync_copy(xt_hbm.at[:, pl.ds(col, CH)], idx_v)

    for b in range(NBUF):
        pltpu.async_copy(table_hbm.at[idx_v.at[b]], rows[b], gsem[b])

    def outer(t, carry):
        for b in range(NBUF):
            h = t * NBUF + b
            pltpu.make_async_copy(
                table_hbm.at[pl.ds(0, CH)], rows[b], gsem[b]).wait()
            pltpu.async_copy(
                rows[b], out_hbm.at[pl.ds(h * BATCH + col, CH)], wsem[b])
            pltpu.make_async_copy(
                rows[b], out_hbm.at[pl.ds(0, CH)], wsem[b]).wait()
            pltpu.async_copy(
                table_hbm.at[idx_v.at[h + NBUF]], rows[b], gsem[b])
        return carry

    lax.fori_loop(0, NCH // NBUF - 1, outer, 0)

    for b in range(NBUF):
        h = NCH - NBUF + b
        pltpu.make_async_copy(
            table_hbm.at[pl.ds(0, CH)], rows[b], gsem[b]).wait()
        pltpu.async_copy(
            rows[b], out_hbm.at[pl.ds(h * BATCH + col, CH)], wsem[b])
    for b in range(NBUF):
        pltpu.make_async_copy(
            rows[b], out_hbm.at[pl.ds(0, CH)], wsem[b]).wait()


def kernel(x, table):
    xt = x.T.astype(jnp.int32)
    out = _emb_gather(xt, table)
    return out.reshape(HIST, BATCH, DIM).transpose(1, 0, 2)


# speedup vs baseline: 1.0036x; 1.0006x over previous
import functools

import jax
import jax.numpy as jnp
from jax import lax
from jax.experimental import pallas as pl
from jax.experimental.pallas import tpu as pltpu
from jax.experimental.pallas import tpu_sc as plsc

VOCAB = 100000
DIM = 128
BATCH = 4096
HIST = 50
N = BATCH * HIST
NC, NS = 2, 16
NW = NC * NS
CH = 128
NCH = HIST
NBUF = 5


@functools.partial(
    pl.kernel,
    mesh=plsc.VectorSubcoreMesh(core_axis_name="c", subcore_axis_name="s"),
    compiler_params=pltpu.CompilerParams(skip_device_barrier=True),
    out_type=jax.ShapeDtypeStruct((N, DIM), jnp.float32),
    scratch_types=[
        pltpu.VMEM((NCH, CH), jnp.int32),
    ]
    + [pltpu.VMEM((CH, DIM), jnp.float32) for _ in range(NBUF)]
    + [pltpu.SemaphoreType.DMA for _ in range(2 * NBUF)],
)
def _emb_gather(xt_hbm, table_hbm, out_hbm, idx_v, *bufs_and_sems):
    rows = bufs_and_sems[:NBUF]
    gsem = bufs_and_sems[NBUF:2 * NBUF]
    wsem = bufs_and_sems[2 * NBUF:]
    wid = lax.axis_index("s") * NC + lax.axis_index("c")
    col = wid * CH
    pltpu.sync_copy(xt_hbm.at[:, pl.ds(col, CH)], idx_v)

    for b in range(NBUF):
        pltpu.async_copy(table_hbm.at[idx_v.at[b]], rows[b], gsem[b])

    def outer(t, carry):
        for b in range(NBUF):
            h = t * NBUF + b
            pltpu.make_async_copy(
                table_hbm.at[pl.ds(0, CH)], rows[b], gsem[b]).wait()
            pltpu.async_copy(
                rows[b], out_hbm.at[pl.ds(h * BATCH + col, CH)], wsem[b])
            pltpu.make_async_copy(
                rows[b], out_hbm.at[pl.ds(0, CH)], wsem[b]).wait()
            pltpu.async_copy(
                table_hbm.at[idx_v.at[h + NBUF]], rows[b], gsem[b])
        return carry

    lax.fori_loop(0, NCH // NBUF - 1, outer, 0)

    for b in range(NBUF):
        h = NCH - NBUF + b
        pltpu.make_async_copy(
            table_hbm.at[pl.ds(0, CH)], rows[b], gsem[b]).wait()
        pltpu.async_copy(
            rows[b], out_hbm.at[pl.ds(h * BATCH + col, CH)], wsem[b])
    for b in range(NBUF):
        pltpu.make_async_copy(
            rows[b], out_hbm.at[pl.ds(0, CH)], wsem[b]).wait()


def kernel(x, table):
    xt = x.T.astype(jnp.int32)
    out = _emb_gather(xt, table)
    return out.reshape(HIST, BATCH, DIM).transpose(1, 0, 2)
